# baseline (speedup 1.0000x reference)
import jax
import jax.numpy as jnp
from jax import lax
from jax.experimental import pallas as pl
from jax.experimental.pallas import tpu as pltpu
from jax.experimental.pallas import tpu_sc as plsc

N = 1_000_000
NW = 32
L = 16
CHUNK = 1_024
NCHUNKS = 977
LAST_START = N - CHUNK
SLOTS = 31


def _rsqrt_nr(a):
    i = lax.bitcast_convert_type(a, jnp.int32)
    i = jnp.int32(0x5F3759DF) - (i >> 1)
    y = lax.bitcast_convert_type(i, jnp.float32)
    ah = 0.5 * a
    for _ in range(3):
        y = y * (1.5 - ah * y * y)
    return y


def _compute_group(b, pos_v, ls_v, rot_v, dr_v, di_v, out_v):
    lane = lax.iota(jnp.int32, L)
    row = b + lane
    cols = [jnp.full((L,), c, jnp.int32) for c in range(16)]

    px = plsc.load_gather(pos_v, [row, cols[0]])
    py = plsc.load_gather(pos_v, [row, cols[1]])
    pz = plsc.load_gather(pos_v, [row, cols[2]])
    sx = jnp.exp(plsc.load_gather(ls_v, [row, cols[0]]))
    sy = jnp.exp(plsc.load_gather(ls_v, [row, cols[1]]))
    sz = jnp.exp(plsc.load_gather(ls_v, [row, cols[2]]))
    qw = plsc.load_gather(rot_v, [row, cols[0]])
    qx = plsc.load_gather(rot_v, [row, cols[1]])
    qy = plsc.load_gather(rot_v, [row, cols[2]])
    qz = plsc.load_gather(rot_v, [row, cols[3]])
    dr = dr_v[pl.ds(b, L)]
    di = di_v[pl.ds(b, L)]

    n2 = qw * qw + qx * qx + qy * qy + qz * qz
    norm = n2 * _rsqrt_nr(n2)
    inv = 1.0 / (norm + 1e-8)
    w, x, y, z = qw * inv, qx * inv, qy * inv, qz * inv

    xx, yy, zz = x * x, y * y, z * z
    xy, xz, yz = x * y, x * z, y * z
    wx, wy, wz = w * x, w * y, w * z
    r00 = 1.0 - 2.0 * (yy + zz)
    r01 = 2.0 * (xy - wz)
    r02 = 2.0 * (xz + wy)
    r10 = 2.0 * (xy + wz)
    r11 = 1.0 - 2.0 * (xx + zz)
    r12 = 2.0 * (yz - wx)
    r20 = 2.0 * (xz - wy)
    r21 = 2.0 * (yz + wx)
    r22 = 1.0 - 2.0 * (xx + yy)

    s2x, s2y, s2z = sx * sx, sy * sy, sz * sz
    a00, a01, a02 = r00 * s2x, r01 * s2y, r02 * s2z
    a10, a11, a12 = r10 * s2x, r11 * s2y, r12 * s2z
    a20, a21, a22 = r20 * s2x, r21 * s2y, r22 * s2z
    s00 = a00 * r00 + a01 * r01 + a02 * r02
    s01 = a00 * r10 + a01 * r11 + a02 * r12
    s02 = a00 * r20 + a01 * r21 + a02 * r22
    s11 = a10 * r10 + a11 * r11 + a12 * r12
    s12 = a10 * r20 + a11 * r21 + a12 * r22
    s22 = a20 * r20 + a21 * r21 + a22 * r22

    t = dr * dr + di * di + 1e-12
    dmag = t * _rsqrt_nr(t)

    vals = (px, py, pz,
            s00, s01, s02, s01, s11, s12, s02, s12, s22,
            sx, sy, sz, dmag)
    for c in range(16):
        plsc.store_scatter(out_v, [row, cols[c]], vals[c])


def _sc_kernel(pos_hbm, ls_hbm, rot_hbm, dr_hbm, di_hbm, out_hbm,
               pos_v, ls_v, rot_v, dr_v, di_v, out_v,
               sem_in, sem_out):
    wid = lax.axis_index("s") * 2 + lax.axis_index("c")

    def chunk_start(slot):
        return jnp.minimum((wid + NW * slot) * CHUNK, LAST_START)

    def in_descs(start, b):
        sl = pl.ds(start, CHUNK)
        return [
            pltpu.make_async_copy(pos_hbm.at[sl], pos_v[b], sem_in[b]),
            pltpu.make_async_copy(ls_hbm.at[sl], ls_v[b], sem_in[b]),
            pltpu.make_async_copy(rot_hbm.at[sl], rot_v[b], sem_in[b]),
            pltpu.make_async_copy(dr_hbm.at[sl], dr_v[b], sem_in[b]),
            pltpu.make_async_copy(di_hbm.at[sl], di_v[b], sem_in[b]),
        ]

    def out_desc(start, b):
        return pltpu.make_async_copy(
            out_v[b], out_hbm.at[pl.ds(start, CHUNK)], sem_out[b])

    def issue_in(slot, b):
        for d in in_descs(chunk_start(slot), b):
            d.start()

    def wait_in(slot, b):
        for d in in_descs(chunk_start(slot), b):
            d.wait()

    issue_in(0, 0)

    def body(t, carry):
        for b in (0, 1):
            g = 2 * t + b

            @pl.when(g < SLOTS)
            def _():
                wait_in(g, b)

                @pl.when(g + 1 < SLOTS)
                def _():
                    issue_in(g + 1, 1 - b)

                @pl.when(g >= 2)
                def _():
                    out_desc(chunk_start(g - 2), b).wait()

                def grp(i, c):
                    _compute_group(i * L, pos_v[b], ls_v[b], rot_v[b],
                                   dr_v[b], di_v[b], out_v[b])
                    return c
                lax.fori_loop(0, CHUNK // L, grp, 0)

                out_desc(chunk_start(g), b).start()
        return carry

    lax.fori_loop(0, (SLOTS + 1) // 2, body, 0)

    out_desc(chunk_start(SLOTS - 2), (SLOTS - 2) % 2).wait()
    out_desc(chunk_start(SLOTS - 1), (SLOTS - 1) % 2).wait()


def kernel(positions, log_scales, rotations, density_real, density_imag):
    mesh = plsc.VectorSubcoreMesh(core_axis_name="c", subcore_axis_name="s")
    f = pl.kernel(
        _sc_kernel,
        out_type=jax.ShapeDtypeStruct((N, 16), jnp.float32),
        mesh=mesh,
        compiler_params=pltpu.CompilerParams(
            needs_layout_passes=False, use_tc_tiling_on_sc=False),
        scratch_types=[
            [pltpu.VMEM((CHUNK, 3), jnp.float32) for _ in range(2)],
            [pltpu.VMEM((CHUNK, 3), jnp.float32) for _ in range(2)],
            [pltpu.VMEM((CHUNK, 4), jnp.float32) for _ in range(2)],
            [pltpu.VMEM((CHUNK,), jnp.float32) for _ in range(2)],
            [pltpu.VMEM((CHUNK,), jnp.float32) for _ in range(2)],
            [pltpu.VMEM((CHUNK, 16), jnp.float32) for _ in range(2)],
            [pltpu.SemaphoreType.DMA for _ in range(2)],
            [pltpu.SemaphoreType.DMA for _ in range(2)],
        ],
    )
    return f(positions, log_scales, rotations, density_real, density_imag)


# speedup vs baseline: 1.1644x; 1.1644x over previous
import jax
import jax.numpy as jnp
from jax import lax
from jax.experimental import pallas as pl
from jax.experimental.pallas import tpu as pltpu
from jax.experimental.pallas import tpu_sc as plsc

N = 1_000_000
NW = 32
L = 16
BLK = 128
CHUNK = 1_024
NBLK = 7_813
MAIN_CHUNKS = 976
TAIL_START = MAIN_CHUNKS * CHUNK
TAIL_IN = N - TAIL_START
TAIL_BLOCKS = 5


def _rsqrt_nr(a):
    i = lax.bitcast_convert_type(a, jnp.int32)
    i = jnp.int32(0x5F3759DF) - (i >> 1)
    y = lax.bitcast_convert_type(i, jnp.float32)
    ah = 0.5 * a
    for _ in range(3):
        y = y * (1.5 - ah * y * y)
    return y


def _compute_group(jj, gg, pos_v, ls_v, rot_v, dr_v, di_v, out_v):
    p0 = jj * BLK + gg * L
    lane = lax.iota(jnp.int32, L)
    row3 = 3 * p0 + 3 * lane
    row4 = 4 * p0 + 4 * lane

    px = plsc.load_gather(pos_v, [row3])
    py = plsc.load_gather(pos_v, [row3 + 1])
    pz = plsc.load_gather(pos_v, [row3 + 2])
    sx = jnp.exp(plsc.load_gather(ls_v, [row3]))
    sy = jnp.exp(plsc.load_gather(ls_v, [row3 + 1]))
    sz = jnp.exp(plsc.load_gather(ls_v, [row3 + 2]))
    qw = plsc.load_gather(rot_v, [row4])
    qx = plsc.load_gather(rot_v, [row4 + 1])
    qy = plsc.load_gather(rot_v, [row4 + 2])
    qz = plsc.load_gather(rot_v, [row4 + 3])
    dr = dr_v[pl.ds(p0, L)]
    di = di_v[pl.ds(p0, L)]

    n2 = qw * qw + qx * qx + qy * qy + qz * qz
    norm = n2 * _rsqrt_nr(n2)
    inv = 1.0 / (norm + 1e-8)
    w, x, y, z = qw * inv, qx * inv, qy * inv, qz * inv

    xx, yy, zz = x * x, y * y, z * z
    xy, xz, yz = x * y, x * z, y * z
    wx, wy, wz = w * x, w * y, w * z
    r00 = 1.0 - 2.0 * (yy + zz)
    r01 = 2.0 * (xy - wz)
    r02 = 2.0 * (xz + wy)
    r10 = 2.0 * (xy + wz)
    r11 = 1.0 - 2.0 * (xx + zz)
    r12 = 2.0 * (yz - wx)
    r20 = 2.0 * (xz - wy)
    r21 = 2.0 * (yz + wx)
    r22 = 1.0 - 2.0 * (xx + yy)

    s2x, s2y, s2z = sx * sx, sy * sy, sz * sz
    a00, a01, a02 = r00 * s2x, r01 * s2y, r02 * s2z
    a10, a11, a12 = r10 * s2x, r11 * s2y, r12 * s2z
    a20, a21, a22 = r20 * s2x, r21 * s2y, r22 * s2z
    s00 = a00 * r00 + a01 * r01 + a02 * r02
    s01 = a00 * r10 + a01 * r11 + a02 * r12
    s02 = a00 * r20 + a01 * r21 + a02 * r22
    s11 = a10 * r10 + a11 * r11 + a12 * r12
    s12 = a10 * r20 + a11 * r21 + a12 * r22
    s22 = a20 * r20 + a21 * r21 + a22 * r22

    t = dr * dr + di * di + 1e-12
    dmag = t * _rsqrt_nr(t)

    vals = (px, py, pz,
            s00, s01, s02, s01, s11, s12, s02, s12, s22,
            sx, sy, sz, dmag)
    l0 = gg * L
    for c in range(16):
        out_v[c // 8, jj, c % 8, pl.ds(l0, L)] = vals[c]


def _make_chunk_fns(pos_hbm, ls_hbm, rot_hbm, dr_hbm, di_hbm, out_hbm,
                    pos_v, ls_v, rot_v, dr_v, di_v, out_v,
                    sem_in, sem_out):
    def in_descs(start, b, npts):
        return [
            pltpu.make_async_copy(pos_hbm.at[pl.ds(start * 3, npts * 3)],
                                  pos_v[b].at[pl.ds(0, npts * 3)], sem_in[b]),
            pltpu.make_async_copy(ls_hbm.at[pl.ds(start * 3, npts * 3)],
                                  ls_v[b].at[pl.ds(0, npts * 3)], sem_in[b]),
            pltpu.make_async_copy(rot_hbm.at[pl.ds(start * 4, npts * 4)],
                                  rot_v[b].at[pl.ds(0, npts * 4)], sem_in[b]),
            pltpu.make_async_copy(dr_hbm.at[pl.ds(start, npts)],
                                  dr_v[b].at[pl.ds(0, npts)], sem_in[b]),
            pltpu.make_async_copy(di_hbm.at[pl.ds(start, npts)],
                                  di_v[b].at[pl.ds(0, npts)], sem_in[b]),
        ]

    def out_descs(j0, b, nblk):
        return [
            pltpu.make_async_copy(out_v[b].at[h, pl.ds(0, nblk)],
                                  out_hbm.at[h, pl.ds(j0, nblk)], sem_out[b])
            for h in (0, 1)
        ]

    def compute(b, nblk):
        def blk_body(jj, c1):
            def grp_body(gg, c2):
                _compute_group(jj, gg, pos_v[b], ls_v[b], rot_v[b],
                               dr_v[b], di_v[b], out_v[b])
                return c2
            lax.fori_loop(0, BLK // L, grp_body, 0)
            return c1
        lax.fori_loop(0, nblk, blk_body, 0)

    return in_descs, out_descs, compute


def _sc_kernel(pos_hbm, ls_hbm, rot_hbm, dr_hbm, di_hbm, out_hbm,
               pos_v, ls_v, rot_v, dr_v, di_v, out_v,
               sem_in, sem_out):
    wid = lax.axis_index("s") * 2 + lax.axis_index("c")
    nslots = 30 + (wid < 16).astype(jnp.int32)

    in_descs, out_descs, compute = _make_chunk_fns(
        pos_hbm, ls_hbm, rot_hbm, dr_hbm, di_hbm, out_hbm,
        pos_v, ls_v, rot_v, dr_v, di_v, out_v, sem_in, sem_out)

    def chunk_start(slot):
        return (wid + NW * slot) * CHUNK

    def issue_in(slot, b, npts):
        for d in in_descs(chunk_start(slot), b, npts):
            d.start()

    def wait_in(slot, b, npts):
        for d in in_descs(chunk_start(slot), b, npts):
            d.wait()

    issue_in(0, 0, CHUNK)

    def body(t, carry):
        for b in (0, 1):
            g = 2 * t + b

            @pl.when(g < nslots)
            def _():
                wait_in(g, b, CHUNK)

                @pl.when(g + 1 < nslots)
                def _():
                    issue_in(g + 1, 1 - b, CHUNK)

                @pl.when(g >= 2)
                def _():
                    for d in out_descs(chunk_start(g - 2) // BLK, b, 8):
                        d.wait()

                compute(b, 8)

                for d in out_descs(chunk_start(g) // BLK, b, 8):
                    d.start()
        return carry

    lax.fori_loop(0, 16, body, 0)

    slot_b0 = jnp.where(nslots == 31, 30, 28)
    for d in out_descs(chunk_start(slot_b0) // BLK, 0, 8):
        d.wait()
    for d in out_descs(chunk_start(29) // BLK, 1, 8):
        d.wait()

    @pl.when(wid == NW - 1)
    def _():
        for d in in_descs(TAIL_START, 0, TAIL_IN):
            d.start()
        for d in in_descs(TAIL_START, 0, TAIL_IN):
            d.wait()
        compute(0, TAIL_BLOCKS)
        for d in out_descs(TAIL_START // BLK, 0, TAIL_BLOCKS):
            d.start()
        for d in out_descs(TAIL_START // BLK, 0, TAIL_BLOCKS):
            d.wait()


def kernel(positions, log_scales, rotations, density_real, density_imag):
    mesh = plsc.VectorSubcoreMesh(core_axis_name="c", subcore_axis_name="s")
    f = pl.kernel(
        _sc_kernel,
        out_type=jax.ShapeDtypeStruct((2, NBLK, 8, BLK), jnp.float32),
        mesh=mesh,
        compiler_params=pltpu.CompilerParams(
            needs_layout_passes=False, use_tc_tiling_on_sc=False),
        scratch_types=[
            [pltpu.VMEM((CHUNK * 3,), jnp.float32) for _ in range(2)],
            [pltpu.VMEM((CHUNK * 3,), jnp.float32) for _ in range(2)],
            [pltpu.VMEM((CHUNK * 4,), jnp.float32) for _ in range(2)],
            [pltpu.VMEM((CHUNK,), jnp.float32) for _ in range(2)],
            [pltpu.VMEM((CHUNK,), jnp.float32) for _ in range(2)],
            [pltpu.VMEM((2, CHUNK // BLK, 8, BLK), jnp.float32)
             for _ in range(2)],
            [pltpu.SemaphoreType.DMA for _ in range(2)],
            [pltpu.SemaphoreType.DMA for _ in range(2)],
        ],
    )
    out4 = f(positions.reshape(-1), log_scales.reshape(-1),
             rotations.reshape(-1), density_real, density_imag)
    out = out4.transpose(1, 3, 0, 2).reshape(NBLK * BLK, 16)
    return out[:N]


# speedup vs baseline: 34.2521x; 29.4150x over previous
import jax
import jax.numpy as jnp
from jax import lax
from jax.experimental import pallas as pl
from jax.experimental.pallas import tpu as pltpu
from jax.experimental.pallas import tpu_sc as plsc

N = 1_000_000
NW = 32
L = 16
BLK = 128
CHUNK = 1_024
NBLK = 7_813
NP = 1_000_448
PBLK = NP // BLK
MAIN_CHUNKS = 976
TAIL_START = MAIN_CHUNKS * CHUNK
TAIL_BLOCKS = 5


def _rsqrt_nr(a):
    i = lax.bitcast_convert_type(a, jnp.int32)
    i = jnp.int32(0x5F3759DF) - (i >> 1)
    y = lax.bitcast_convert_type(i, jnp.float32)
    ah = 0.5 * a
    for _ in range(3):
        y = y * (1.5 - ah * y * y)
    return y


def _compute_group(jj, gg, pos_v, ls_v, rot_v, dr_v, di_v, out_v):
    p0 = jj * BLK + gg * L
    sl = pl.ds(p0, L)

    px = pos_v[0, sl]
    py = pos_v[1, sl]
    pz = pos_v[2, sl]
    sx = jnp.exp(ls_v[0, sl])
    sy = jnp.exp(ls_v[1, sl])
    sz = jnp.exp(ls_v[2, sl])
    qw = rot_v[0, sl]
    qx = rot_v[1, sl]
    qy = rot_v[2, sl]
    qz = rot_v[3, sl]
    dr = dr_v[sl]
    di = di_v[sl]

    n2 = qw * qw + qx * qx + qy * qy + qz * qz
    norm = n2 * _rsqrt_nr(n2)
    inv = 1.0 / (norm + 1e-8)
    w, x, y, z = qw * inv, qx * inv, qy * inv, qz * inv

    xx, yy, zz = x * x, y * y, z * z
    xy, xz, yz = x * y, x * z, y * z
    wx, wy, wz = w * x, w * y, w * z
    r00 = 1.0 - 2.0 * (yy + zz)
    r01 = 2.0 * (xy - wz)
    r02 = 2.0 * (xz + wy)
    r10 = 2.0 * (xy + wz)
    r11 = 1.0 - 2.0 * (xx + zz)
    r12 = 2.0 * (yz - wx)
    r20 = 2.0 * (xz - wy)
    r21 = 2.0 * (yz + wx)
    r22 = 1.0 - 2.0 * (xx + yy)

    s2x, s2y, s2z = sx * sx, sy * sy, sz * sz
    a00, a01, a02 = r00 * s2x, r01 * s2y, r02 * s2z
    a10, a11, a12 = r10 * s2x, r11 * s2y, r12 * s2z
    a20, a21, a22 = r20 * s2x, r21 * s2y, r22 * s2z
    s00 = a00 * r00 + a01 * r01 + a02 * r02
    s01 = a00 * r10 + a01 * r11 + a02 * r12
    s02 = a00 * r20 + a01 * r21 + a02 * r22
    s11 = a10 * r10 + a11 * r11 + a12 * r12
    s12 = a10 * r20 + a11 * r21 + a12 * r22
    s22 = a20 * r20 + a21 * r21 + a22 * r22

    t = dr * dr + di * di + 1e-12
    dmag = t * _rsqrt_nr(t)

    vals = (px, py, pz,
            s00, s01, s02, s01, s11, s12, s02, s12, s22,
            sx, sy, sz, dmag)
    l0 = gg * L
    for c in range(16):
        out_v[c // 8, jj, c % 8, pl.ds(l0, L)] = vals[c]


def _make_chunk_fns(pos_hbm, ls_hbm, rot_hbm, dr_hbm, di_hbm, out_hbm,
                    pos_v, ls_v, rot_v, dr_v, di_v, out_v,
                    sem_in, sem_out):
    def in_descs(j0, b, nblk, npts_d):
        npts = nblk * BLK
        descs = []
        for src, dst, nc in ((pos_hbm, pos_v[b], 3), (ls_hbm, ls_v[b], 3),
                             (rot_hbm, rot_v[b], 4)):
            for c in range(nc):
                descs.append(pltpu.make_async_copy(
                    src.at[pl.ds(c * NP + j0 * BLK, npts)],
                    dst.at[c, pl.ds(0, npts)], sem_in[b]))
        descs.append(pltpu.make_async_copy(
            dr_hbm.at[pl.ds(j0 * BLK, npts_d)],
            dr_v[b].at[pl.ds(0, npts_d)], sem_in[b]))
        descs.append(pltpu.make_async_copy(
            di_hbm.at[pl.ds(j0 * BLK, npts_d)],
            di_v[b].at[pl.ds(0, npts_d)], sem_in[b]))
        return descs

    def out_descs(j0, b, nblk):
        return [
            pltpu.make_async_copy(out_v[b].at[h, pl.ds(0, nblk)],
                                  out_hbm.at[h, pl.ds(j0, nblk)], sem_out[b])
            for h in (0, 1)
        ]

    def compute(b, nblk):
        def blk_body(jj, c1):
            def grp_body(gg, c2):
                _compute_group(jj, gg, pos_v[b], ls_v[b], rot_v[b],
                               dr_v[b], di_v[b], out_v[b])
                return c2
            lax.fori_loop(0, BLK // L, grp_body, 0)
            return c1
        lax.fori_loop(0, nblk, blk_body, 0)

    return in_descs, out_descs, compute


def _sc_kernel(pos_hbm, ls_hbm, rot_hbm, dr_hbm, di_hbm, out_hbm,
               pos_v, ls_v, rot_v, dr_v, di_v, out_v,
               sem_in, sem_out):
    wid = lax.axis_index("s") * 2 + lax.axis_index("c")
    nslots = 30 + (wid < 16).astype(jnp.int32)

    in_descs, out_descs, compute = _make_chunk_fns(
        pos_hbm, ls_hbm, rot_hbm, dr_hbm, di_hbm, out_hbm,
        pos_v, ls_v, rot_v, dr_v, di_v, out_v, sem_in, sem_out)

    def chunk_j0(slot):
        return (wid + NW * slot) * (CHUNK // BLK)

    def issue_in(slot, b):
        for d in in_descs(chunk_j0(slot), b, CHUNK // BLK, CHUNK):
            d.start()

    def wait_in(slot, b):
        for d in in_descs(chunk_j0(slot), b, CHUNK // BLK, CHUNK):
            d.wait()

    issue_in(0, 0)

    def body(t, carry):
        for b in (0, 1):
            g = 2 * t + b

            @pl.when(g < nslots)
            def _():
                wait_in(g, b)

                @pl.when(g + 1 < nslots)
                def _():
                    issue_in(g + 1, 1 - b)

                @pl.when(g >= 2)
                def _():
                    for d in out_descs(chunk_j0(g - 2), b, 8):
                        d.wait()

                compute(b, 8)

                for d in out_descs(chunk_j0(g), b, 8):
                    d.start()
        return carry

    lax.fori_loop(0, 16, body, 0)

    slot_b0 = jnp.where(nslots == 31, 30, 28)
    for d in out_descs(chunk_j0(slot_b0), 0, 8):
        d.wait()
    for d in out_descs(chunk_j0(29), 1, 8):
        d.wait()

    @pl.when(wid == NW - 1)
    def _():
        j0 = TAIL_START // BLK
        for d in in_descs(j0, 0, TAIL_BLOCKS, N - TAIL_START):
            d.start()
        for d in in_descs(j0, 0, TAIL_BLOCKS, N - TAIL_START):
            d.wait()
        compute(0, TAIL_BLOCKS)
        for d in out_descs(j0, 0, TAIL_BLOCKS):
            d.start()
        for d in out_descs(j0, 0, TAIL_BLOCKS):
            d.wait()


def kernel(positions, log_scales, rotations, density_real, density_imag):
    mesh = plsc.VectorSubcoreMesh(core_axis_name="c", subcore_axis_name="s")
    f = pl.kernel(
        _sc_kernel,
        out_type=jax.ShapeDtypeStruct((2, NBLK, 8, BLK), jnp.float32),
        mesh=mesh,
        compiler_params=pltpu.CompilerParams(
            needs_layout_passes=False, use_tc_tiling_on_sc=False),
        scratch_types=[
            [pltpu.VMEM((3, CHUNK), jnp.float32) for _ in range(2)],
            [pltpu.VMEM((3, CHUNK), jnp.float32) for _ in range(2)],
            [pltpu.VMEM((4, CHUNK), jnp.float32) for _ in range(2)],
            [pltpu.VMEM((CHUNK,), jnp.float32) for _ in range(2)],
            [pltpu.VMEM((CHUNK,), jnp.float32) for _ in range(2)],
            [pltpu.VMEM((2, CHUNK // BLK, 8, BLK), jnp.float32)
             for _ in range(2)],
            [pltpu.SemaphoreType.DMA for _ in range(2)],
            [pltpu.SemaphoreType.DMA for _ in range(2)],
        ],
    )

    def planar(a, ncols):
        ap = jnp.pad(a, ((0, NP - N), (0, 0)))
        return ap.T.reshape(ncols * PBLK * BLK)

    out4 = f(planar(positions, 3), planar(log_scales, 3),
             planar(rotations, 4), density_real, density_imag)
    out = out4.transpose(1, 3, 0, 2).reshape(NBLK * BLK, 16)
    return out[:N]


# speedup vs baseline: 34.8446x; 1.0173x over previous
import jax
import jax.numpy as jnp
from jax import lax
from jax.experimental import pallas as pl
from jax.experimental.pallas import tpu as pltpu
from jax.experimental.pallas import tpu_sc as plsc

N = 1_000_000
NW = 32
L = 16
BLK = 128
CHUNK = 2_048
NBLK = 7_813
NP = 1_000_448
PBLK = NP // BLK
MAIN_CHUNKS = 488
TAIL_START = MAIN_CHUNKS * CHUNK
TAIL_BLOCKS = 5


def _rsqrt_nr(a):
    i = lax.bitcast_convert_type(a, jnp.int32)
    i = jnp.int32(0x5F3759DF) - (i >> 1)
    y = lax.bitcast_convert_type(i, jnp.float32)
    ah = 0.5 * a
    for _ in range(3):
        y = y * (1.5 - ah * y * y)
    return y


def _compute_group(jj, gg, pos_v, ls_v, rot_v, dr_v, di_v, out_v):
    p0 = jj * BLK + gg * L
    sl = pl.ds(p0, L)

    px = pos_v[0, sl]
    py = pos_v[1, sl]
    pz = pos_v[2, sl]
    sx = jnp.exp(ls_v[0, sl])
    sy = jnp.exp(ls_v[1, sl])
    sz = jnp.exp(ls_v[2, sl])
    qw = rot_v[0, sl]
    qx = rot_v[1, sl]
    qy = rot_v[2, sl]
    qz = rot_v[3, sl]
    dr = dr_v[sl]
    di = di_v[sl]

    n2 = qw * qw + qx * qx + qy * qy + qz * qz
    norm = n2 * _rsqrt_nr(n2)
    inv = 1.0 / (norm + 1e-8)
    w, x, y, z = qw * inv, qx * inv, qy * inv, qz * inv

    xx, yy, zz = x * x, y * y, z * z
    xy, xz, yz = x * y, x * z, y * z
    wx, wy, wz = w * x, w * y, w * z
    r00 = 1.0 - 2.0 * (yy + zz)
    r01 = 2.0 * (xy - wz)
    r02 = 2.0 * (xz + wy)
    r10 = 2.0 * (xy + wz)
    r11 = 1.0 - 2.0 * (xx + zz)
    r12 = 2.0 * (yz - wx)
    r20 = 2.0 * (xz - wy)
    r21 = 2.0 * (yz + wx)
    r22 = 1.0 - 2.0 * (xx + yy)

    s2x, s2y, s2z = sx * sx, sy * sy, sz * sz
    a00, a01, a02 = r00 * s2x, r01 * s2y, r02 * s2z
    a10, a11, a12 = r10 * s2x, r11 * s2y, r12 * s2z
    a20, a21, a22 = r20 * s2x, r21 * s2y, r22 * s2z
    s00 = a00 * r00 + a01 * r01 + a02 * r02
    s01 = a00 * r10 + a01 * r11 + a02 * r12
    s02 = a00 * r20 + a01 * r21 + a02 * r22
    s11 = a10 * r10 + a11 * r11 + a12 * r12
    s12 = a10 * r20 + a11 * r21 + a12 * r22
    s22 = a20 * r20 + a21 * r21 + a22 * r22

    t = dr * dr + di * di + 1e-12
    dmag = t * _rsqrt_nr(t)

    vals = (px, py, pz,
            s00, s01, s02, s01, s11, s12, s02, s12, s22,
            sx, sy, sz, dmag)
    l0 = gg * L
    for c in range(16):
        out_v[c // 8, jj, c % 8, pl.ds(l0, L)] = vals[c]


def _make_chunk_fns(pos_hbm, ls_hbm, rot_hbm, dr_hbm, di_hbm, out_hbm,
                    pos_v, ls_v, rot_v, dr_v, di_v, out_v,
                    sem_in, sem_out):
    def in_descs(j0, b, nblk, npts_d):
        npts = nblk * BLK
        descs = []
        for src, dst, nc in ((pos_hbm, pos_v[b], 3), (ls_hbm, ls_v[b], 3),
                             (rot_hbm, rot_v[b], 4)):
            for c in range(nc):
                descs.append(pltpu.make_async_copy(
                    src.at[pl.ds(c * NP + j0 * BLK, npts)],
                    dst.at[c, pl.ds(0, npts)], sem_in[b]))
        descs.append(pltpu.make_async_copy(
            dr_hbm.at[pl.ds(j0 * BLK, npts_d)],
            dr_v[b].at[pl.ds(0, npts_d)], sem_in[b]))
        descs.append(pltpu.make_async_copy(
            di_hbm.at[pl.ds(j0 * BLK, npts_d)],
            di_v[b].at[pl.ds(0, npts_d)], sem_in[b]))
        return descs

    def out_descs(j0, b, nblk):
        return [
            pltpu.make_async_copy(out_v[b].at[h, pl.ds(0, nblk)],
                                  out_hbm.at[h, pl.ds(j0, nblk)], sem_out[b])
            for h in (0, 1)
        ]

    def compute(b, nblk):
        def blk_body(jj, c1):
            for gg in range(BLK // L):
                _compute_group(jj, gg, pos_v[b], ls_v[b], rot_v[b],
                               dr_v[b], di_v[b], out_v[b])
            return c1
        lax.fori_loop(0, nblk, blk_body, 0)

    return in_descs, out_descs, compute


def _sc_kernel(pos_hbm, ls_hbm, rot_hbm, dr_hbm, di_hbm, out_hbm,
               pos_v, ls_v, rot_v, dr_v, di_v, out_v,
               sem_in, sem_out):
    wid = lax.axis_index("s") * 2 + lax.axis_index("c")
    nslots = 15 + (wid < 8).astype(jnp.int32)

    in_descs, out_descs, compute = _make_chunk_fns(
        pos_hbm, ls_hbm, rot_hbm, dr_hbm, di_hbm, out_hbm,
        pos_v, ls_v, rot_v, dr_v, di_v, out_v, sem_in, sem_out)

    def chunk_j0(slot):
        return (wid + NW * slot) * (CHUNK // BLK)

    def issue_in(slot, b):
        for d in in_descs(chunk_j0(slot), b, CHUNK // BLK, CHUNK):
            d.start()

    def wait_in(slot, b):
        for d in in_descs(chunk_j0(slot), b, CHUNK // BLK, CHUNK):
            d.wait()

    issue_in(0, 0)

    def body(t, carry):
        for b in (0, 1):
            g = 2 * t + b

            @pl.when(g < nslots)
            def _():
                wait_in(g, b)

                @pl.when(g + 1 < nslots)
                def _():
                    issue_in(g + 1, 1 - b)

                @pl.when(g >= 2)
                def _():
                    for d in out_descs(chunk_j0(g - 2), b, CHUNK // BLK):
                        d.wait()

                compute(b, CHUNK // BLK)

                for d in out_descs(chunk_j0(g), b, CHUNK // BLK):
                    d.start()
        return carry

    lax.fori_loop(0, 8, body, 0)

    slot_b0 = jnp.where(nslots == 16, 14, 14)
    slot_b1 = jnp.where(nslots == 16, 15, 13)
    for d in out_descs(chunk_j0(slot_b0), 0, CHUNK // BLK):
        d.wait()
    for d in out_descs(chunk_j0(slot_b1), 1, CHUNK // BLK):
        d.wait()

    @pl.when(wid == NW - 1)
    def _():
        j0 = TAIL_START // BLK
        for d in in_descs(j0, 0, TAIL_BLOCKS, N - TAIL_START):
            d.start()
        for d in in_descs(j0, 0, TAIL_BLOCKS, N - TAIL_START):
            d.wait()
        compute(0, TAIL_BLOCKS)
        for d in out_descs(j0, 0, TAIL_BLOCKS):
            d.start()
        for d in out_descs(j0, 0, TAIL_BLOCKS):
            d.wait()


def kernel(positions, log_scales, rotations, density_real, density_imag):
    mesh = plsc.VectorSubcoreMesh(core_axis_name="c", subcore_axis_name="s")
    f = pl.kernel(
        _sc_kernel,
        out_type=jax.ShapeDtypeStruct((2, NBLK, 8, BLK), jnp.float32),
        mesh=mesh,
        compiler_params=pltpu.CompilerParams(
            needs_layout_passes=False, use_tc_tiling_on_sc=False),
        scratch_types=[
            [pltpu.VMEM((3, CHUNK), jnp.float32) for _ in range(2)],
            [pltpu.VMEM((3, CHUNK), jnp.float32) for _ in range(2)],
            [pltpu.VMEM((4, CHUNK), jnp.float32) for _ in range(2)],
            [pltpu.VMEM((CHUNK,), jnp.float32) for _ in range(2)],
            [pltpu.VMEM((CHUNK,), jnp.float32) for _ in range(2)],
            [pltpu.VMEM((2, CHUNK // BLK, 8, BLK), jnp.float32)
             for _ in range(2)],
            [pltpu.SemaphoreType.DMA for _ in range(2)],
            [pltpu.SemaphoreType.DMA for _ in range(2)],
        ],
    )

    def planar(a, ncols):
        ap = jnp.pad(a, ((0, NP - N), (0, 0)))
        return ap.T.reshape(ncols * PBLK * BLK)

    out4 = f(planar(positions, 3), planar(log_scales, 3),
             planar(rotations, 4), density_real, density_imag)
    out = out4.transpose(1, 3, 0, 2).reshape(NBLK * BLK, 16)
    return out[:N]
